# baseline (speedup 1.0000x reference)
import functools

import jax
import jax.numpy as jnp
from jax import lax
from jax.experimental import pallas as pl
from jax.experimental.pallas import tpu as pltpu
from jax.experimental.pallas import tpu_sc as plsc

N = 10000
E = 320000
C = 128
NW = 32
EPW = E // NW
EB = 80
NCH = EPW // EB
EPS = E // 16
NR = 10
NB = EPS // (NR * EB)
RPS = N // 16

_MESH = plsc.VectorSubcoreMesh(core_axis_name="c", subcore_axis_name="s")
_F32 = jnp.float32
_SC_PARAMS = pltpu.CompilerParams(needs_layout_passes=False)


def _norm_body(idx_ref, w_ref, norm_ref, deg_o, deg_i, idxb, wb, nb, dob, dib):
    c = lax.axis_index("c")
    s = lax.axis_index("s")
    zv = jnp.zeros((16,), _F32)

    @pl.when(s < 2)
    def _zero():
        def zb_body(j, carry):
            dob[pl.ds(j * 16, 16)] = zv
            return carry
        lax.fori_loop(0, N // 16, zb_body, 0)

    @pl.when(s == 0)
    def _z0():
        pltpu.sync_copy(dob, deg_o)

    @pl.when(s == 1)
    def _z1():
        pltpu.sync_copy(dob, deg_i)

    plsc.subcore_barrier()

    def deg_body(i, carry):
        pltpu.sync_copy(w_ref.at[c, s, i], wb)
        pltpu.sync_copy(idx_ref.at[c, 0, s, i], idxb)
        for r in range(NR):
            pltpu.sync_copy(wb.at[r], deg_o.at[idxb.at[r]], add=True)
        pltpu.sync_copy(idx_ref.at[c, 1, s, i], idxb)
        for r in range(NR):
            pltpu.sync_copy(wb.at[r], deg_i.at[idxb.at[r]], add=True)
        return carry

    lax.fori_loop(0, NB, deg_body, 0)
    plsc.subcore_barrier()

    pltpu.sync_copy(deg_o, dob)
    pltpu.sync_copy(deg_i, dib)

    def norm_chunk(i, carry):
        pltpu.sync_copy(w_ref.at[c, s, i], wb)
        for d, db in ((0, dob), (1, dib)):
            pltpu.sync_copy(idx_ref.at[c, d, s, i], idxb)
            for r in range(NR):
                for j in range(EB // 16):
                    sl = pl.ds(j * 16, 16)
                    nb[r, sl] = wb[r, sl] / plsc.load_gather(db, [idxb[r, sl]])
            pltpu.sync_copy(nb, norm_ref.at[c, d, s, i])
        return carry

    lax.fori_loop(0, NB, norm_chunk, 0)


_norm_kernel = pl.kernel(
    _norm_body,
    out_type=jax.ShapeDtypeStruct((2, 2, 16, NB, NR, EB), _F32),
    mesh=_MESH,
    compiler_params=_SC_PARAMS,
    scratch_types=[
        pltpu.VMEM_SHARED((N,), _F32),
        pltpu.VMEM_SHARED((N,), _F32),
        pltpu.VMEM((NR, EB), jnp.int32),
        pltpu.VMEM((NR, EB), _F32),
        pltpu.VMEM((NR, EB), _F32),
        pltpu.VMEM((N,), _F32),
        pltpu.VMEM((N,), _F32),
    ],
)


ZR = 25


def _prop_core(dual, feat_ref, src_ref, dst_ref, cf_ref, out_ref,
               acc, zb, idxs, idxd, cfb, msg0, msg1, gsem, ssem):
    c = lax.axis_index("c")
    s = lax.axis_index("s")
    fr = feat_ref.at[c] if dual else feat_ref
    bufs = (msg0, msg1)

    zv = jnp.zeros((16,), _F32)

    def zb_body(i, carry):
        for h in range(C // 16):
            zb[i, pl.ds(h * 16, 16)] = zv
        return carry

    lax.fori_loop(0, ZR, zb_body, 0)
    for q in range(RPS // ZR):
        pltpu.sync_copy(zb, acc.at[pl.ds(s * RPS + q * ZR, ZR)])
    plsc.subcore_barrier()

    def scale(mb, r):
        @plsc.parallel_loop(0, EB // 16, unroll=1)
        def _scale_body(g):
            cfv = cfb[r, pl.ds(g * 16, 16)]
            for j in range(16):
                cf16 = lax.gather(
                    cfv, jnp.full((16, 1), j, jnp.int32),
                    lax.GatherDimensionNumbers(offset_dims=(),
                                               collapsed_slice_dims=(0,),
                                               start_index_map=(0,)),
                    (1,), mode=lax.GatherScatterMode.PROMISE_IN_BOUNDS)
                e = g * 16 + j
                for h in range(C // 16):
                    sl = pl.ds(h * 16, 16)
                    mb[e, sl] = mb[e, sl] * cf16

    def chunk(i, carry):
        pltpu.sync_copy(src_ref.at[c, s, i], idxs)
        pltpu.sync_copy(dst_ref.at[c, s, i], idxd)
        pltpu.sync_copy(cf_ref.at[c, s, i], cfb)
        gh = pltpu.async_copy(fr.at[idxs.at[0]], bufs[0], gsem)
        sh = [None, None]
        for r in range(NR):
            mb = bufs[r % 2]
            gh.wait()
            if r + 1 < NR:
                if sh[(r + 1) % 2] is not None:
                    sh[(r + 1) % 2].wait()
                    sh[(r + 1) % 2] = None
                gh = pltpu.async_copy(fr.at[idxs.at[r + 1]], bufs[(r + 1) % 2],
                                      gsem)
            scale(mb, r)
            sh[r % 2] = pltpu.async_copy(mb, acc.at[idxd.at[r]], ssem, add=True)
        for b in range(2):
            if sh[b] is not None:
                sh[b].wait()
        return carry

    lax.fori_loop(0, NB, chunk, 0)
    plsc.subcore_barrier()
    pltpu.sync_copy(acc.at[pl.ds(s * RPS, RPS)], out_ref.at[c, s])


_PROP_SCRATCH = [
    pltpu.VMEM_SHARED((N, C), _F32),
    pltpu.VMEM((ZR, C), _F32),
    pltpu.VMEM((NR, EB), jnp.int32),
    pltpu.VMEM((NR, EB), jnp.int32),
    pltpu.VMEM((NR, EB), _F32),
    pltpu.VMEM((EB, C), _F32),
    pltpu.VMEM((EB, C), _F32),
    pltpu.SemaphoreType.DMA,
    pltpu.SemaphoreType.DMA,
]

_prop_kernel = pl.kernel(
    functools.partial(_prop_core, False),
    out_type=jax.ShapeDtypeStruct((2, 16, RPS, C), _F32),
    mesh=_MESH,
    compiler_params=_SC_PARAMS,
    scratch_types=_PROP_SCRATCH,
)

_prop_kernel_dual = pl.kernel(
    functools.partial(_prop_core, True),
    out_type=jax.ShapeDtypeStruct((2, 16, RPS, C), _F32),
    mesh=_MESH,
    compiler_params=_SC_PARAMS,
    scratch_types=_PROP_SCRATCH,
)


def _final_s_body(x_ref, t1o_ref, t1i_ref, q2o_ref, q2i_ref, wst_ref, b_ref, o_ref):
    x = x_ref[...]
    t2o = 2.0 * q2o_ref[...] - x
    t2i = 2.0 * q2i_ref[...] - x
    g = jnp.dot(x, wst_ref[0], preferred_element_type=_F32)
    g += jnp.dot(t1o_ref[...], wst_ref[1], preferred_element_type=_F32)
    g += jnp.dot(t1i_ref[...], wst_ref[2], preferred_element_type=_F32)
    g += jnp.dot(t2o, wst_ref[3], preferred_element_type=_F32)
    g += jnp.dot(t2i, wst_ref[4], preferred_element_type=_F32)
    g += b_ref[...]
    z = jax.nn.sigmoid(g[:, :C])
    o_ref[...] = (1.0 - z) * jnp.tanh(g[:, C:])


def _final_p_body(x_ref, r1o_ref, r1i_ref, wst_ref, b_ref, o_ref):
    x = x_ref[...]
    g = jnp.dot(x, wst_ref[0], preferred_element_type=_F32)
    g += jnp.dot(r1o_ref[...], wst_ref[1], preferred_element_type=_F32)
    g += jnp.dot(r1i_ref[...], wst_ref[2], preferred_element_type=_F32)
    g += b_ref[...]
    z = jax.nn.sigmoid(g[:, :C])
    o_ref[...] = (1.0 - z) * jnp.tanh(g[:, C:])


_R = 1000


def _stack_weights(Wz, Wh, bz, bh, K):
    Wz = Wz[:, :, :C, :]
    Wh = Wh[:, :, :C, :]
    mats = [jnp.concatenate([Wz[0, 0] + Wz[1, 0], Wh[0, 0] + Wh[1, 0]], axis=1)]
    for k in range(1, K):
        mats.append(jnp.concatenate([Wz[0, k], Wh[0, k]], axis=1))
        mats.append(jnp.concatenate([Wz[1, k], Wh[1, k]], axis=1))
    return jnp.stack(mats), jnp.concatenate([bz, bh]).reshape(1, 2 * C)


def kernel(X_s, X_p, edge_index_s, edge_weight_s, edge_index_p, edge_weight_p,
           edge_index_p2s, edge_weight_p2s,
           W_z_s, b_z_s, W_z_p, b_z_p, W_r_s, b_r_s, W_r_p, b_r_p,
           W_h_s, b_h_s, W_h_p, b_h_p):
    del edge_index_p2s, edge_weight_p2s, W_r_s, b_r_s, W_r_p, b_r_p
    Xs = X_s[0]
    Xp = X_p[0]

    idx_all = jnp.stack([edge_index_s, edge_index_p]).reshape(2, 2, 16, NB, NR, EB)
    w_all = jnp.stack([edge_weight_s, edge_weight_p]).reshape(2, 16, NB, NR, EB)
    norms = _norm_kernel(idx_all, w_all).reshape(2, 2, E)

    shp = (16, NB, NR, EB)
    row_s = edge_index_s[0].reshape(shp)
    col_s = edge_index_s[1].reshape(shp)
    row_p = edge_index_p[0].reshape(shp)
    col_p = edge_index_p[1].reshape(shp)
    src_s = jnp.stack([row_s, col_s])
    dst_s = jnp.stack([col_s, row_s])
    src_p = jnp.stack([row_p, col_p])
    dst_p = jnp.stack([col_p, row_p])
    cf_s = norms[0].reshape((2,) + shp)
    cf_p = norms[1].reshape((2,) + shp)

    t1 = _prop_kernel(Xs, src_s, dst_s, cf_s).reshape(2, N, C)
    r1 = _prop_kernel(Xp, src_p, dst_p, cf_p).reshape(2, N, C)
    t1o, t1i = t1[0], t1[1]
    q2 = _prop_kernel_dual(t1, src_s, dst_s, cf_s).reshape(2, N, C)
    q2o, q2i = q2[0], q2[1]

    Ws, bs = _stack_weights(W_z_s, W_h_s, b_z_s, b_h_s, 3)
    Wp, bp = _stack_weights(W_z_p, W_h_p, b_z_p, b_h_p, 2)

    blk = pl.BlockSpec((_R, C), lambda i: (i, 0))

    out_s = pl.pallas_call(
        _final_s_body,
        grid=(N // _R,),
        in_specs=[blk, blk, blk, blk, blk,
                  pl.BlockSpec((5, C, 2 * C), lambda i: (0, 0, 0)),
                  pl.BlockSpec((1, 2 * C), lambda i: (0, 0))],
        out_specs=blk,
        out_shape=jax.ShapeDtypeStruct((N, C), _F32),
    )(Xs, t1o, t1i, q2o, q2i, Ws, bs)

    out_p = pl.pallas_call(
        _final_p_body,
        grid=(N // _R,),
        in_specs=[blk, blk, blk,
                  pl.BlockSpec((3, C, 2 * C), lambda i: (0, 0, 0)),
                  pl.BlockSpec((1, 2 * C), lambda i: (0, 0))],
        out_specs=blk,
        out_shape=jax.ShapeDtypeStruct((N, C), _F32),
    )(Xp, r1[0], r1[1], Wp, bp)

    return jnp.concatenate([out_s, out_p], axis=0)[None]


# speedup vs baseline: 1.4598x; 1.4598x over previous
import functools

import jax
import jax.numpy as jnp
from jax import lax
from jax.experimental import pallas as pl
from jax.experimental.pallas import tpu as pltpu
from jax.experimental.pallas import tpu_sc as plsc

N = 10000
E = 320000
C = 128
NW = 32
EPW = E // NW
EB = 80
NCH = EPW // EB
EPS = E // 16
NR = 10
NB = EPS // (NR * EB)
RPS = N // 16

_MESH = plsc.VectorSubcoreMesh(core_axis_name="c", subcore_axis_name="s")
_F32 = jnp.float32
_SC_PARAMS = pltpu.CompilerParams(needs_layout_passes=False)


def _norm_body(idx_ref, w_ref, norm_ref, deg_o, deg_i, idxb, wb, nb, dob, dib):
    c = lax.axis_index("c")
    s = lax.axis_index("s")
    zv = jnp.zeros((16,), _F32)

    @pl.when(s < 2)
    def _zero():
        def zb_body(j, carry):
            dob[pl.ds(j * 16, 16)] = zv
            return carry
        lax.fori_loop(0, N // 16, zb_body, 0)

    @pl.when(s == 0)
    def _z0():
        pltpu.sync_copy(dob, deg_o)

    @pl.when(s == 1)
    def _z1():
        pltpu.sync_copy(dob, deg_i)

    plsc.subcore_barrier()

    def deg_body(i, carry):
        pltpu.sync_copy(w_ref.at[c, s, i], wb)
        pltpu.sync_copy(idx_ref.at[c, 0, s, i], idxb)
        for r in range(NR):
            pltpu.sync_copy(wb.at[r], deg_o.at[idxb.at[r]], add=True)
        pltpu.sync_copy(idx_ref.at[c, 1, s, i], idxb)
        for r in range(NR):
            pltpu.sync_copy(wb.at[r], deg_i.at[idxb.at[r]], add=True)
        return carry

    lax.fori_loop(0, NB, deg_body, 0)
    plsc.subcore_barrier()

    pltpu.sync_copy(deg_o, dob)
    pltpu.sync_copy(deg_i, dib)

    def norm_chunk(i, carry):
        pltpu.sync_copy(w_ref.at[c, s, i], wb)
        for d, db in ((0, dob), (1, dib)):
            pltpu.sync_copy(idx_ref.at[c, d, s, i], idxb)
            for r in range(NR):
                for j in range(EB // 16):
                    sl = pl.ds(j * 16, 16)
                    nb[r, sl] = wb[r, sl] / plsc.load_gather(db, [idxb[r, sl]])
            pltpu.sync_copy(nb, norm_ref.at[c, d, s, i])
        return carry

    lax.fori_loop(0, NB, norm_chunk, 0)


_norm_kernel = pl.kernel(
    _norm_body,
    out_type=jax.ShapeDtypeStruct((2, 2, 16, NB, NR, EB), _F32),
    mesh=_MESH,
    compiler_params=_SC_PARAMS,
    scratch_types=[
        pltpu.VMEM_SHARED((N,), _F32),
        pltpu.VMEM_SHARED((N,), _F32),
        pltpu.VMEM((NR, EB), jnp.int32),
        pltpu.VMEM((NR, EB), _F32),
        pltpu.VMEM((NR, EB), _F32),
        pltpu.VMEM((N,), _F32),
        pltpu.VMEM((N,), _F32),
    ],
)


ZR = 25


def _prop_core(dual, feat_ref, src_ref, dst_ref, cf_ref, out_ref,
               acc, zb, idxs0, idxd0, cfb0, idxs1, idxd1, cfb1,
               msg0, msg1, msg2, gsem, ssem, isem):
    c = lax.axis_index("c")
    s = lax.axis_index("s")
    fr = feat_ref.at[c] if dual else feat_ref
    bufs = (msg0, msg1, msg2)

    zv = jnp.zeros((16,), _F32)

    def zb_body(i, carry):
        for h in range(C // 16):
            zb[i, pl.ds(h * 16, 16)] = zv
        return carry

    lax.fori_loop(0, ZR, zb_body, 0)
    for q in range(RPS // ZR):
        pltpu.sync_copy(zb, acc.at[pl.ds(s * RPS + q * ZR, ZR)])
    plsc.subcore_barrier()

    def scale(mb, r, cfb):
        @plsc.parallel_loop(0, EB, unroll=4)
        def _scale_body(e):
            cf16 = plsc.load_gather(cfb, [jnp.full((16,), r, jnp.int32),
                                          jnp.full((16,), e, jnp.int32)])
            for h in range(C // 16):
                sl = pl.ds(h * 16, 16)
                mb[e, sl] = mb[e, sl] * cf16

    def prefetch(i, idxs, idxd, cfb):
        pltpu.async_copy(src_ref.at[c, s, i], idxs, isem)
        pltpu.async_copy(dst_ref.at[c, s, i], idxd, isem)
        pltpu.async_copy(cf_ref.at[c, s, i], cfb, isem)

    def rows(idxs, idxd, cfb):
        pltpu.make_async_copy(src_ref.at[c, s, 0], idxs, isem).wait()
        pltpu.make_async_copy(dst_ref.at[c, s, 0], idxd, isem).wait()
        pltpu.make_async_copy(cf_ref.at[c, s, 0], cfb, isem).wait()
        gh = [None, None, None]
        sh = [None, None, None]
        gh[0] = pltpu.async_copy(fr.at[idxs.at[0]], bufs[0], gsem)
        gh[1] = pltpu.async_copy(fr.at[idxs.at[1]], bufs[1], gsem)
        for r in range(NR):
            b = r % 3
            gh[b].wait()
            if r + 2 < NR:
                nb2 = (r + 2) % 3
                if sh[nb2] is not None:
                    sh[nb2].wait()
                    sh[nb2] = None
                gh[nb2] = pltpu.async_copy(fr.at[idxs.at[r + 2]], bufs[nb2],
                                           gsem)
            scale(bufs[b], r, cfb)
            sh[b] = pltpu.async_copy(bufs[b], acc.at[idxd.at[r]], ssem,
                                     add=True)
        for b in range(3):
            if sh[b] is not None:
                sh[b].wait()

    prefetch(0, idxs0, idxd0, cfb0)
    prefetch(1, idxs1, idxd1, cfb1)

    def pair_body(t, carry):
        i0 = 2 * t
        rows(idxs0, idxd0, cfb0)
        prefetch(i0 + 2, idxs0, idxd0, cfb0)
        rows(idxs1, idxd1, cfb1)

        @pl.when(i0 + 3 < NB)
        def _pf():
            prefetch(i0 + 3, idxs1, idxd1, cfb1)

        return carry

    lax.fori_loop(0, (NB - 1) // 2, pair_body, 0)
    rows(idxs0, idxd0, cfb0)
    plsc.subcore_barrier()
    pltpu.sync_copy(acc.at[pl.ds(s * RPS, RPS)], out_ref.at[c, s])


_PROP_SCRATCH = [
    pltpu.VMEM_SHARED((N, C), _F32),
    pltpu.VMEM((ZR, C), _F32),
    pltpu.VMEM((NR, EB), jnp.int32),
    pltpu.VMEM((NR, EB), jnp.int32),
    pltpu.VMEM((NR, EB), _F32),
    pltpu.VMEM((NR, EB), jnp.int32),
    pltpu.VMEM((NR, EB), jnp.int32),
    pltpu.VMEM((NR, EB), _F32),
    pltpu.VMEM((EB, C), _F32),
    pltpu.VMEM((EB, C), _F32),
    pltpu.VMEM((EB, C), _F32),
    pltpu.SemaphoreType.DMA,
    pltpu.SemaphoreType.DMA,
    pltpu.SemaphoreType.DMA,
]

_prop_kernel = pl.kernel(
    functools.partial(_prop_core, False),
    out_type=jax.ShapeDtypeStruct((2, 16, RPS, C), _F32),
    mesh=_MESH,
    compiler_params=_SC_PARAMS,
    scratch_types=_PROP_SCRATCH,
)

_prop_kernel_dual = pl.kernel(
    functools.partial(_prop_core, True),
    out_type=jax.ShapeDtypeStruct((2, 16, RPS, C), _F32),
    mesh=_MESH,
    compiler_params=_SC_PARAMS,
    scratch_types=_PROP_SCRATCH,
)


def _final_s_body(x_ref, t1o_ref, t1i_ref, q2o_ref, q2i_ref, wst_ref, b_ref, o_ref):
    x = x_ref[...]
    t2o = 2.0 * q2o_ref[...] - x
    t2i = 2.0 * q2i_ref[...] - x
    g = jnp.dot(x, wst_ref[0], preferred_element_type=_F32)
    g += jnp.dot(t1o_ref[...], wst_ref[1], preferred_element_type=_F32)
    g += jnp.dot(t1i_ref[...], wst_ref[2], preferred_element_type=_F32)
    g += jnp.dot(t2o, wst_ref[3], preferred_element_type=_F32)
    g += jnp.dot(t2i, wst_ref[4], preferred_element_type=_F32)
    g += b_ref[...]
    z = jax.nn.sigmoid(g[:, :C])
    o_ref[...] = (1.0 - z) * jnp.tanh(g[:, C:])


def _final_p_body(x_ref, r1o_ref, r1i_ref, wst_ref, b_ref, o_ref):
    x = x_ref[...]
    g = jnp.dot(x, wst_ref[0], preferred_element_type=_F32)
    g += jnp.dot(r1o_ref[...], wst_ref[1], preferred_element_type=_F32)
    g += jnp.dot(r1i_ref[...], wst_ref[2], preferred_element_type=_F32)
    g += b_ref[...]
    z = jax.nn.sigmoid(g[:, :C])
    o_ref[...] = (1.0 - z) * jnp.tanh(g[:, C:])


_R = 1000


def _stack_weights(Wz, Wh, bz, bh, K):
    Wz = Wz[:, :, :C, :]
    Wh = Wh[:, :, :C, :]
    mats = [jnp.concatenate([Wz[0, 0] + Wz[1, 0], Wh[0, 0] + Wh[1, 0]], axis=1)]
    for k in range(1, K):
        mats.append(jnp.concatenate([Wz[0, k], Wh[0, k]], axis=1))
        mats.append(jnp.concatenate([Wz[1, k], Wh[1, k]], axis=1))
    return jnp.stack(mats), jnp.concatenate([bz, bh]).reshape(1, 2 * C)


def kernel(X_s, X_p, edge_index_s, edge_weight_s, edge_index_p, edge_weight_p,
           edge_index_p2s, edge_weight_p2s,
           W_z_s, b_z_s, W_z_p, b_z_p, W_r_s, b_r_s, W_r_p, b_r_p,
           W_h_s, b_h_s, W_h_p, b_h_p):
    del edge_index_p2s, edge_weight_p2s, W_r_s, b_r_s, W_r_p, b_r_p
    Xs = X_s[0]
    Xp = X_p[0]

    idx_all = jnp.stack([edge_index_s, edge_index_p]).reshape(2, 2, 16, NB, NR, EB)
    w_all = jnp.stack([edge_weight_s, edge_weight_p]).reshape(2, 16, NB, NR, EB)
    norms = _norm_kernel(idx_all, w_all).reshape(2, 2, E)

    shp = (16, NB, NR, EB)
    row_s = edge_index_s[0].reshape(shp)
    col_s = edge_index_s[1].reshape(shp)
    row_p = edge_index_p[0].reshape(shp)
    col_p = edge_index_p[1].reshape(shp)
    src_s = jnp.stack([row_s, col_s])
    dst_s = jnp.stack([col_s, row_s])
    src_p = jnp.stack([row_p, col_p])
    dst_p = jnp.stack([col_p, row_p])
    cf_s = norms[0].reshape((2,) + shp)
    cf_p = norms[1].reshape((2,) + shp)

    t1 = _prop_kernel(Xs, src_s, dst_s, cf_s).reshape(2, N, C)
    r1 = _prop_kernel(Xp, src_p, dst_p, cf_p).reshape(2, N, C)
    t1o, t1i = t1[0], t1[1]
    q2 = _prop_kernel_dual(t1, src_s, dst_s, cf_s).reshape(2, N, C)
    q2o, q2i = q2[0], q2[1]

    Ws, bs = _stack_weights(W_z_s, W_h_s, b_z_s, b_h_s, 3)
    Wp, bp = _stack_weights(W_z_p, W_h_p, b_z_p, b_h_p, 2)

    blk = pl.BlockSpec((_R, C), lambda i: (i, 0))

    out_s = pl.pallas_call(
        _final_s_body,
        grid=(N // _R,),
        in_specs=[blk, blk, blk, blk, blk,
                  pl.BlockSpec((5, C, 2 * C), lambda i: (0, 0, 0)),
                  pl.BlockSpec((1, 2 * C), lambda i: (0, 0))],
        out_specs=blk,
        out_shape=jax.ShapeDtypeStruct((N, C), _F32),
    )(Xs, t1o, t1i, q2o, q2i, Ws, bs)

    out_p = pl.pallas_call(
        _final_p_body,
        grid=(N // _R,),
        in_specs=[blk, blk, blk,
                  pl.BlockSpec((3, C, 2 * C), lambda i: (0, 0, 0)),
                  pl.BlockSpec((1, 2 * C), lambda i: (0, 0))],
        out_specs=blk,
        out_shape=jax.ShapeDtypeStruct((N, C), _F32),
    )(Xp, r1[0], r1[1], Wp, bp)

    return jnp.concatenate([out_s, out_p], axis=0)[None]
